# Initial kernel scaffold; baseline (speedup 1.0000x reference)
import jax
import jax.numpy as jnp
from jax.experimental import pallas as pl


def kernel(input1, input2):
    raise NotImplementedError("write your pallas kernel here")


# speedup vs baseline: 6.3623x; 6.3623x over previous
import functools

import jax
import jax.numpy as jnp
from jax import lax
from jax.experimental import pallas as pl
from jax.experimental.pallas import tpu as pltpu
from jax.experimental.pallas import tpu_sc as plsc


def _split_pixel(p, B, H, W, HW):
    if (W & (W - 1)) == 0 and (HW & (HW - 1)) == 0:
        x = p & (W - 1)
        y = (p >> (W.bit_length() - 1)) & (H - 1)
        boff = p - (p & (HW - 1))
    else:
        x = p % W
        q = p // W
        y = q % H
        boff = (q // H) * HW
    return boff, y, x


@functools.lru_cache(maxsize=None)
def _build_warp(B, C, H, W):
    HW = H * W
    N = B * HW
    info = plsc.get_sparse_core_info()
    NC = info.num_cores
    NW = NC * info.num_subcores
    L = info.num_lanes
    K = 128
    assert N % NW == 0 and C % L == 0
    PPW = N // NW
    assert PPW % K == 0
    NCHUNK = PPW // K

    mesh = plsc.VectorSubcoreMesh(core_axis_name="core", subcore_axis_name="sub")

    @functools.partial(
        pl.kernel,
        out_type=jax.ShapeDtypeStruct((N, C), jnp.float32),
        mesh=mesh,
        compiler_params=pltpu.CompilerParams(use_tc_tiling_on_sc=False),
        scratch_types=[
            pltpu.VMEM((K,), jnp.float32),
            pltpu.VMEM((K,), jnp.float32),
            pltpu.VMEM((K,), jnp.int32),
            pltpu.VMEM((K,), jnp.int32),
            pltpu.VMEM((K,), jnp.int32),
            pltpu.VMEM((K,), jnp.int32),
            pltpu.VMEM((K,), jnp.float32),
            pltpu.VMEM((K,), jnp.float32),
            pltpu.VMEM((K,), jnp.float32),
            pltpu.VMEM((K,), jnp.float32),
            pltpu.VMEM((K, C), jnp.float32),
            pltpu.VMEM((K, C), jnp.float32),
            pltpu.VMEM((K, C), jnp.float32),
            pltpu.VMEM((K, C), jnp.float32),
            pltpu.VMEM((K, C), jnp.float32),
            pltpu.SemaphoreType.DMA,
        ],
    )
    def warp(table, fx_hbm, fy_hbm, out_hbm,
             fxv, fyv, i00, i01, i10, i11, w00, w01, w10, w11,
             r00, r01, r10, r11, outv, sem):
        wid = lax.axis_index("sub") * NC + lax.axis_index("core")
        base = wid * PPW
        lanes = lax.iota(jnp.int32, L)

        def chunk(ci, carry):
            p0 = base + ci * K
            pltpu.sync_copy(fx_hbm.at[pl.ds(p0, K)], fxv)
            pltpu.sync_copy(fy_hbm.at[pl.ds(p0, K)], fyv)

            for g in range(K // L):
                s = g * L
                p = p0 + s + lanes
                boff, yi, xi = _split_pixel(p, B, H, W, HW)
                xf = xi.astype(jnp.float32) + fxv[pl.ds(s, L)]
                yf = yi.astype(jnp.float32) + fyv[pl.ds(s, L)]
                xfc = jnp.clip(xf, -1.0, float(W))
                yfc = jnp.clip(yf, -1.0, float(H))
                xt = xfc.astype(jnp.int32)
                yt = yfc.astype(jnp.int32)
                x0i = jnp.where(xt.astype(jnp.float32) > xfc, xt - 1, xt)
                y0i = jnp.where(yt.astype(jnp.float32) > yfc, yt - 1, yt)
                a = xfc - x0i.astype(jnp.float32)
                b = yfc - y0i.astype(jnp.float32)
                x0 = jnp.clip(x0i, 0, W - 1)
                x1 = jnp.clip(x0i + 1, 0, W - 1)
                y0 = jnp.clip(y0i, 0, H - 1)
                y1 = jnp.clip(y0i + 1, 0, H - 1)
                r0 = boff + y0 * W
                r1 = boff + y1 * W
                sl = pl.ds(s, L)
                i00[sl] = r0 + x0
                i01[sl] = r0 + x1
                i10[sl] = r1 + x0
                i11[sl] = r1 + x1
                ia = 1.0 - a
                ib = 1.0 - b
                w00[sl] = ia * ib
                w01[sl] = a * ib
                w10[sl] = ia * b
                w11[sl] = a * b

            d0 = pltpu.async_copy(table.at[i00], r00, sem)
            d1 = pltpu.async_copy(table.at[i01], r01, sem)
            d2 = pltpu.async_copy(table.at[i10], r10, sem)
            d3 = pltpu.async_copy(table.at[i11], r11, sem)
            d0.wait()
            d1.wait()
            d2.wait()
            d3.wait()

            def grp2(g, c2):
                s2 = g * L
                wv00 = w00[pl.ds(s2, L)]
                wv01 = w01[pl.ds(s2, L)]
                wv10 = w10[pl.ds(s2, L)]
                wv11 = w11[pl.ds(s2, L)]
                for kk in range(L):
                    k = s2 + kk
                    a00 = wv00[kk]
                    a01 = wv01[kk]
                    a10 = wv10[kk]
                    a11 = wv11[kk]
                    for j in range(C // L):
                        cs = pl.ds(j * L, L)
                        outv[k, cs] = (a00 * r00[k, cs] + a01 * r01[k, cs]
                                       + a10 * r10[k, cs] + a11 * r11[k, cs])
                return c2

            lax.fori_loop(0, K // L, grp2, 0)
            pltpu.sync_copy(outv, out_hbm.at[pl.ds(p0, K)])
            return carry

        lax.fori_loop(0, NCHUNK, chunk, 0)

    return warp


def kernel(input1, input2):
    B, C, H, W = input1.shape
    table = input1.transpose(0, 2, 3, 1).reshape(B * H * W, C)
    fx = input2[:, 0].reshape(-1)
    fy = input2[:, 1].reshape(-1)
    out = _build_warp(B, C, H, W)(table, fx, fy)
    return out.reshape(B, H, W, C).transpose(0, 3, 1, 2)
